# Initial kernel scaffold; baseline (speedup 1.0000x reference)
import jax
import jax.numpy as jnp
from jax.experimental import pallas as pl


def kernel(x, table):
    raise NotImplementedError("write your pallas kernel here")


# speedup vs baseline: 3.5498x; 3.5498x over previous
import functools

import jax
import jax.numpy as jnp
from jax import lax
from jax.experimental import pallas as pl
from jax.experimental.pallas import tpu as pltpu
from jax.experimental.pallas import tpu_sc as plsc

EMBED = 64
NC = 2
NS = 16
NW = NC * NS
CH = 128


@functools.partial(jax.jit, static_argnums=(2, 3))
def _gather(idx, table, n_per_w, n_chunks):
    @functools.partial(
        pl.kernel,
        out_type=jax.ShapeDtypeStruct((n_per_w * NW, EMBED), jnp.float32),
        mesh=plsc.VectorSubcoreMesh(core_axis_name="c", subcore_axis_name="s"),
        scratch_types=[
            pltpu.VMEM((n_per_w,), jnp.int32),
            pltpu.VMEM((CH, EMBED), jnp.float32),
            pltpu.SemaphoreType.DMA,
        ],
        compiler_params=pltpu.CompilerParams(use_tc_tiling_on_sc=False),
    )
    def k(idx_hbm, table_hbm, out_hbm, idx_v, rows_v, sem):
        wid = lax.axis_index("s") * NC + lax.axis_index("c")
        base = pl.multiple_of(wid * n_per_w, CH)
        pltpu.sync_copy(idx_hbm.at[pl.ds(base, n_per_w)], idx_v)

        def body(g, carry):
            off = pl.multiple_of(g * CH, CH)
            pltpu.async_copy(
                table_hbm.at[idx_v.at[pl.ds(off, CH)]], rows_v, sem
            ).wait()
            pltpu.sync_copy(rows_v, out_hbm.at[pl.ds(base + off, CH)])
            return carry

        lax.fori_loop(0, n_chunks, body, 0)

    return k(idx, table)


def kernel(x, table):
    b, s = x.shape
    n = b * s
    idx = x.reshape(n).astype(jnp.int32)
    n_per_w = n // NW
    out = _gather(idx, table, n_per_w, n_per_w // CH)
    return out.reshape(b, s, EMBED)


# speedup vs baseline: 4.1923x; 1.1810x over previous
import functools

import jax
import jax.numpy as jnp
from jax import lax
from jax.experimental import pallas as pl
from jax.experimental.pallas import tpu as pltpu
from jax.experimental.pallas import tpu_sc as plsc

EMBED = 64
NC = 2
NS = 16
NW = NC * NS
CH = 1024


@functools.partial(jax.jit, static_argnums=(2, 3))
def _gather(idx, table, n_per_w, n_chunks):
    @functools.partial(
        pl.kernel,
        out_type=jax.ShapeDtypeStruct((n_per_w * NW, EMBED), jnp.float32),
        mesh=plsc.VectorSubcoreMesh(core_axis_name="c", subcore_axis_name="s"),
        scratch_types=[
            pltpu.VMEM((n_per_w,), jnp.int32),
            pltpu.VMEM((CH, EMBED), jnp.float32),
            pltpu.SemaphoreType.DMA,
        ],
        compiler_params=pltpu.CompilerParams(use_tc_tiling_on_sc=False),
    )
    def k(idx_hbm, table_hbm, out_hbm, idx_v, rows_v, sem):
        wid = lax.axis_index("s") * NC + lax.axis_index("c")
        base = pl.multiple_of(wid * n_per_w, CH)
        pltpu.sync_copy(idx_hbm.at[pl.ds(base, n_per_w)], idx_v)

        def body(g, carry):
            off = pl.multiple_of(g * CH, CH)
            pltpu.async_copy(
                table_hbm.at[idx_v.at[pl.ds(off, CH)]], rows_v, sem
            ).wait()
            pltpu.sync_copy(rows_v, out_hbm.at[pl.ds(base + off, CH)])
            return carry

        lax.fori_loop(0, n_chunks, body, 0)

    return k(idx, table)


def kernel(x, table):
    b, s = x.shape
    n = b * s
    idx = x.reshape(n).astype(jnp.int32)
    n_per_w = n // NW
    out = _gather(idx, table, n_per_w, n_per_w // CH)
    return out.reshape(b, s, EMBED)


# speedup vs baseline: 4.2598x; 1.0161x over previous
import functools

import jax
import jax.numpy as jnp
from jax import lax
from jax.experimental import pallas as pl
from jax.experimental.pallas import tpu as pltpu
from jax.experimental.pallas import tpu_sc as plsc

EMBED = 64
NC = 2
NS = 16
NW = NC * NS
CH = 640


@functools.partial(jax.jit, static_argnums=(2, 3))
def _gather(idx, table, n_per_w, n_chunks):
    n_super = n_chunks // 2

    @functools.partial(
        pl.kernel,
        out_type=jax.ShapeDtypeStruct((n_per_w * NW, EMBED), jnp.float32),
        mesh=plsc.VectorSubcoreMesh(core_axis_name="c", subcore_axis_name="s"),
        scratch_types=[
            pltpu.VMEM((n_per_w,), jnp.int32),
            pltpu.VMEM((CH, EMBED), jnp.float32),
            pltpu.VMEM((CH, EMBED), jnp.float32),
            pltpu.SemaphoreType.DMA,
            pltpu.SemaphoreType.DMA,
            pltpu.SemaphoreType.DMA,
            pltpu.SemaphoreType.DMA,
        ],
        compiler_params=pltpu.CompilerParams(use_tc_tiling_on_sc=False),
    )
    def k(idx_hbm, table_hbm, out_hbm, idx_v, rows0, rows1, g0, g1, o0, o1):
        rows = (rows0, rows1)
        gsem = (g0, g1)
        osem = (o0, o1)
        wid = lax.axis_index("s") * NC + lax.axis_index("c")
        base = pl.multiple_of(wid * n_per_w, CH)
        pltpu.sync_copy(idx_hbm.at[pl.ds(base, n_per_w)], idx_v)

        def idx_slice(g):
            return idx_v.at[pl.ds(pl.multiple_of(g * CH, CH), CH)]

        def out_slice(g):
            return out_hbm.at[pl.ds(base + pl.multiple_of(g * CH, CH), CH)]

        def gather_start(g, s):
            pltpu.async_copy(table_hbm.at[idx_slice(g)], rows[s], gsem[s])

        def gather_wait(s):
            pltpu.make_async_copy(
                table_hbm.at[pl.ds(0, CH)], rows[s], gsem[s]
            ).wait()

        def out_start(g, s):
            pltpu.async_copy(rows[s], out_slice(g), osem[s])

        def out_wait(s):
            pltpu.make_async_copy(rows[s], out_hbm.at[pl.ds(0, CH)], osem[s]).wait()

        gather_start(0, 0)

        def body(t, carry):
            for j in (0, 1):
                g = t * 2 + j
                sn = (j + 1) % 2
                if j == 0:
                    @pl.when(t >= 1)
                    def _():
                        out_wait(sn)
                    gather_start(g + 1, sn)
                else:
                    out_wait(sn)

                    @pl.when(t < n_super - 1)
                    def _():
                        gather_start(g + 1, sn)
                gather_wait(j)
                out_start(g, j)
            return carry

        lax.fori_loop(0, n_super, body, 0)
        out_wait(1)

    return k(idx, table)


def kernel(x, table):
    b, s = x.shape
    n = b * s
    idx = x.reshape(n).astype(jnp.int32)
    n_per_w = n // NW
    out = _gather(idx, table, n_per_w, n_per_w // CH)
    return out.reshape(b, s, EMBED)
